# baseline (speedup 1.0000x reference)
import functools

import jax
import jax.numpy as jnp
from jax import lax
from jax.experimental import pallas as pl
from jax.experimental.pallas import tpu as pltpu
from jax.experimental.pallas import tpu_sc as plsc

F32 = jnp.float32
BF16 = jnp.bfloat16
I32 = jnp.int32

K_TOP = 32
LN_EPS = 1e-12

B, T, H = 4, 2048, 768
NM = 512
ROWS, VPR, KD = 1024, 64, 128
MSIZE = ROWS * VPR
VD = 128
FLAT = B * T
NW = 32
RK = 8192
NBLK = MSIZE // RK
RPB = RK // VPR
ETILE = 256
NTILE = FLAT // ETILE

def _wid():
    return lax.axis_index("s") * 2 + lax.axis_index("c")


@functools.cache
def _sc_gather_se():
    @functools.partial(
        pl.kernel,
        mesh=plsc.VectorSubcoreMesh(core_axis_name="c", subcore_axis_name="s"),
        out_type=[jax.ShapeDtypeStruct((NM, H), F32),
                  jax.ShapeDtypeStruct((NM, H), F32)],
        scratch_types=[pltpu.VMEM((16,), I32),
                       pltpu.VMEM((16,), I32),
                       pltpu.VMEM((16,), I32),
                       pltpu.VMEM((16, H), F32),
                       pltpu.VMEM((16, H), F32),
                       pltpu.SemaphoreType.DMA],
    )
    def k(flat_hbm, bpos_hbm, spos_hbm, epos_hbm, out_s, out_e,
          bidx_v, idx_v, idx2_v, rows_v, rows2_v, sem):
        base = _wid() * 16
        pltpu.sync_copy(bpos_hbm.at[pl.ds(base, 16)], bidx_v)
        pltpu.sync_copy(spos_hbm.at[pl.ds(base, 16)], idx_v)
        pltpu.sync_copy(epos_hbm.at[pl.ds(base, 16)], idx2_v)
        idx_v[...] = bidx_v[...] * T + idx_v[...]
        idx2_v[...] = bidx_v[...] * T + idx2_v[...]
        c1 = pltpu.async_copy(flat_hbm.at[idx_v], rows_v, sem)
        c2 = pltpu.async_copy(flat_hbm.at[idx2_v], rows2_v, sem)
        c1.wait()
        c2.wait()
        pltpu.sync_copy(rows_v, out_s.at[pl.ds(base, 16)])
        pltpu.sync_copy(rows2_v, out_e.at[pl.ds(base, 16)])

    return k


@functools.cache
def _sc_gather_topk():
    @functools.partial(
        pl.kernel,
        mesh=plsc.VectorSubcoreMesh(core_axis_name="c", subcore_axis_name="s"),
        out_type=[jax.ShapeDtypeStruct((NM * K_TOP, VD), F32),
                  jax.ShapeDtypeStruct((NM * K_TOP,), I32)],
        scratch_types=[pltpu.VMEM((512,), I32),
                       pltpu.VMEM((512, VD), F32),
                       pltpu.VMEM((512,), I32),
                       pltpu.SemaphoreType.DMA,
                       pltpu.SemaphoreType.DMA],
    )
    def k(tid_hbm, vals_hbm, eids_hbm, out_v, out_e,
          idx_v, rows_v, eid_v, sem, sem2):
        w = _wid()
        base = w * 512
        pltpu.sync_copy(tid_hbm.at[pl.ds(base, 512)], idx_v)
        cps = []
        for c in range(4):
            sl = pl.ds(c * 128, 128)
            cps.append(pltpu.async_copy(vals_hbm.at[idx_v.at[sl]],
                                        rows_v.at[sl], sem))
            cps.append(pltpu.async_copy(eids_hbm.at[idx_v.at[sl]],
                                        eid_v.at[sl], sem2))
        for cp in cps:
            cp.wait()
        pltpu.sync_copy(rows_v, out_v.at[pl.ds(base, 512)])
        pltpu.sync_copy(eid_v, out_e.at[pl.ds(base, 512)])

    return k


def _abc_body(s_ref, e_ref, wq_ref, bq_ref, k_ref,
              q_out, ts_ref, ti_ref, at_ref,
              q_s, rm_s, code_s):
    pid = pl.program_id(0)

    @pl.when(pid == 0)
    def _():
        cat = jnp.concatenate((s_ref[...], e_ref[...]), axis=-1).astype(BF16)
        wb = wq_ref[...].astype(BF16)
        acc = None
        for i in range(6):
            c = lax.dot_general(cat[:, 256 * i:256 * (i + 1)],
                                wb[256 * i:256 * (i + 1)],
                                (((1,), (0,)), ((), ())),
                                preferred_element_type=F32)
            acc = c if acc is None else acc + c
        q = acc + bq_ref[...]
        q_out[...] = q
        q_s[...] = q

    kb = k_ref[...].astype(BF16)
    qb = q_s[...].astype(BF16)
    st = lax.dot_general(kb, qb, (((1,), (1,)), ((), ())),
                         preferred_element_type=F32)
    s3 = st.reshape(RPB, VPR, NM)
    m = jnp.max(s3, axis=1)
    iot = lax.broadcasted_iota(I32, (RPB, VPR, NM), 1)
    a = jnp.min(jnp.where(s3 == m[:, None, :], iot, VPR), axis=1)
    rowg = pid * RPB + lax.broadcasted_iota(I32, (RPB, NM), 0)
    rm_s[pl.ds(pid * RPB, RPB), :] = m
    code_s[pl.ds(pid * RPB, RPB), :] = rowg * VPR + a

    @pl.when(pid == NBLK - 1)
    def _():
        work = rm_s[...]
        code = code_s[...]
        ts_rows = []
        for k in range(4):
            mx = jnp.max(work, axis=0, keepdims=True)
            tid = jnp.min(jnp.where(work == mx, code, MSIZE),
                          axis=0, keepdims=True)
            hit = (code >> 6) == (tid >> 6)
            ts_ref[k:k + 1, :] = mx
            ti_ref[k:k + 1, :] = tid
            work = jnp.where(hit, -jnp.inf, work)
            ts_rows.append(mx)
        ts_ref[4:K_TOP, :] = jnp.zeros((K_TOP - 4, NM), F32)
        ti_ref[4:K_TOP, :] = jnp.zeros((K_TOP - 4, NM), I32)
        ts = jnp.concatenate(ts_rows * 8, axis=0)
        ex = jnp.exp(ts - ts[0:1, :])
        at_ref[...] = ex / jnp.sum(ex, axis=0, keepdims=True)


def _e_body(att_ref, v_ref, wu_ref, bu_ref, m_ref,
            enc_ref, bp_ref, sp_ref, ep_ref, g_ref, b_ref,
            o_ref, proj_s):
    pid = pl.program_id(0)

    @pl.when(pid == 0)
    def _():
        v3 = v_ref[...].reshape(NM, K_TOP, VD)
        pooled = jnp.sum(v3 * att_ref[...][:, :, None], axis=1)
        proj = lax.dot_general(
            pooled.astype(BF16), wu_ref[...].astype(BF16),
            (((1,), (0,)), ((), ())), preferred_element_type=F32) + bu_ref[...]
        proj_s[...] = (proj * m_ref[...]).astype(BF16)

    base = pid * ETILE
    r = base + lax.broadcasted_iota(I32, (ETILE, 1), 0)
    ps = bp_ref[...] * T + sp_ref[...]
    pe = bp_ref[...] * T + ep_ref[...]
    mhot = ((r == ps).astype(F32) + (r == pe).astype(F32)).astype(BF16)
    delta = lax.dot_general(mhot, proj_s[...], (((1,), (0,)), ((), ())),
                            preferred_element_type=F32)
    x = enc_ref[...] + delta
    mean = jnp.mean(x, axis=-1, keepdims=True)
    xc = x - mean
    var = jnp.mean(xc * xc, axis=-1, keepdims=True)
    o_ref[...] = xc * lax.rsqrt(var + LN_EPS) * g_ref[...] + b_ref[...]


def _tc_abc(start_enc, end_enc, W_query, b_query, kflat):
    return pl.pallas_call(
        _abc_body,
        grid=(NBLK,),
        in_specs=[pl.BlockSpec((NM, H), lambda k: (0, 0)),
                  pl.BlockSpec((NM, H), lambda k: (0, 0)),
                  pl.BlockSpec((2 * H, KD), lambda k: (0, 0)),
                  pl.BlockSpec((1, KD), lambda k: (0, 0)),
                  pl.BlockSpec((RK, KD), lambda k: (k, 0))],
        out_specs=[pl.BlockSpec((NM, KD), lambda k: (0, 0)),
                   pl.BlockSpec((K_TOP, NM), lambda k: (0, 0)),
                   pl.BlockSpec((K_TOP, NM), lambda k: (0, 0)),
                   pl.BlockSpec((K_TOP, NM), lambda k: (0, 0))],
        out_shape=[jax.ShapeDtypeStruct((NM, KD), F32),
                   jax.ShapeDtypeStruct((K_TOP, NM), F32),
                   jax.ShapeDtypeStruct((K_TOP, NM), I32),
                   jax.ShapeDtypeStruct((K_TOP, NM), F32)],
        scratch_shapes=[pltpu.VMEM((NM, KD), F32),
                        pltpu.VMEM((ROWS, NM), F32),
                        pltpu.VMEM((ROWS, NM), I32)],
    )(start_enc, end_enc, W_query, b_query.reshape(1, KD), kflat)


def _tc_e(attn, values_g, W_update, b_update, mask_f, encf, bp, sp, ep,
          ln_scale, ln_bias):
    return pl.pallas_call(
        _e_body,
        grid=(NTILE,),
        in_specs=[pl.BlockSpec((NM, K_TOP), lambda k: (0, 0)),
                  pl.BlockSpec((NM * K_TOP, VD), lambda k: (0, 0)),
                  pl.BlockSpec((VD, H), lambda k: (0, 0)),
                  pl.BlockSpec((1, H), lambda k: (0, 0)),
                  pl.BlockSpec((NM, 1), lambda k: (0, 0)),
                  pl.BlockSpec((ETILE, H), lambda k: (k, 0)),
                  pl.BlockSpec((1, NM), lambda k: (0, 0)),
                  pl.BlockSpec((1, NM), lambda k: (0, 0)),
                  pl.BlockSpec((1, NM), lambda k: (0, 0)),
                  pl.BlockSpec((1, H), lambda k: (0, 0)),
                  pl.BlockSpec((1, H), lambda k: (0, 0))],
        out_specs=pl.BlockSpec((ETILE, H), lambda k: (k, 0)),
        out_shape=jax.ShapeDtypeStruct((FLAT, H), F32),
        scratch_shapes=[pltpu.VMEM((NM, H), BF16)],
    )(attn, values_g, W_update, b_update.reshape(1, H), mask_f,
      encf, bp, sp, ep, ln_scale.reshape(1, H), ln_bias.reshape(1, H))


def kernel(encoded_input, mention_batch_positions, mention_start_positions,
           mention_end_positions, mention_mask, memory_keys, memory_identifiers,
           memory_entity_ids, memory_values, W_query, b_query, W_update, b_update,
           ln_scale, ln_bias):
    encf = encoded_input.reshape(FLAT, H)
    kflat = memory_keys.reshape(MSIZE, KD)

    start_enc, end_enc = _sc_gather_se()(
        encf, mention_batch_positions, mention_start_positions,
        mention_end_positions)
    queries, ts, ti, attn_t = _tc_abc(start_enc, end_enc, W_query, b_query,
                                      kflat)

    tid_flat = ti.T.reshape(NM * K_TOP)
    values_g, eids = _sc_gather_topk()(tid_flat, memory_values,
                                       memory_entity_ids)

    attn = attn_t.T
    mask_f = mention_mask.astype(F32).reshape(NM, 1)
    enc_out = _tc_e(attn, values_g, W_update, b_update, mask_f, encf,
                    mention_batch_positions.reshape(1, NM),
                    mention_start_positions.reshape(1, NM),
                    mention_end_positions.reshape(1, NM),
                    ln_scale, ln_bias)

    return (enc_out.reshape(B, T, H), queries, attn,
            eids.reshape(NM, K_TOP))


# speedup vs baseline: 4.6648x; 4.6648x over previous
import functools

import jax
import jax.numpy as jnp
from jax import lax
from jax.experimental import pallas as pl
from jax.experimental.pallas import tpu as pltpu
from jax.experimental.pallas import tpu_sc as plsc

F32 = jnp.float32
BF16 = jnp.bfloat16
I32 = jnp.int32

K_TOP = 32
LN_EPS = 1e-12

B, T, H = 4, 2048, 768
NM = 512
ROWS, VPR, KD = 1024, 64, 128
MSIZE = ROWS * VPR
VD = 128
FLAT = B * T
NW = 32
RK = 4096
NBLK = MSIZE // RK
RPB = RK // VPR
ETILE = 256
NTILE = FLAT // ETILE

def _wid():
    return lax.axis_index("s") * 2 + lax.axis_index("c")


@functools.cache
def _sc_gather_se():
    @functools.partial(
        pl.kernel,
        mesh=plsc.VectorSubcoreMesh(core_axis_name="c", subcore_axis_name="s"),
        out_type=[jax.ShapeDtypeStruct((NM, H), F32),
                  jax.ShapeDtypeStruct((NM, H), F32)],
        scratch_types=[pltpu.VMEM((16,), I32),
                       pltpu.VMEM((16,), I32),
                       pltpu.VMEM((16,), I32),
                       pltpu.VMEM((16, H), F32),
                       pltpu.VMEM((16, H), F32),
                       pltpu.SemaphoreType.DMA],
    )
    def k(flat_hbm, bpos_hbm, spos_hbm, epos_hbm, out_s, out_e,
          bidx_v, idx_v, idx2_v, rows_v, rows2_v, sem):
        base = _wid() * 16
        pltpu.sync_copy(bpos_hbm.at[pl.ds(base, 16)], bidx_v)
        pltpu.sync_copy(spos_hbm.at[pl.ds(base, 16)], idx_v)
        pltpu.sync_copy(epos_hbm.at[pl.ds(base, 16)], idx2_v)
        idx_v[...] = bidx_v[...] * T + idx_v[...]
        idx2_v[...] = bidx_v[...] * T + idx2_v[...]
        c1 = pltpu.async_copy(flat_hbm.at[idx_v], rows_v, sem)
        c2 = pltpu.async_copy(flat_hbm.at[idx2_v], rows2_v, sem)
        c1.wait()
        c2.wait()
        pltpu.sync_copy(rows_v, out_s.at[pl.ds(base, 16)])
        pltpu.sync_copy(rows2_v, out_e.at[pl.ds(base, 16)])

    return k


@functools.cache
def _sc_gather_topk():
    @functools.partial(
        pl.kernel,
        mesh=plsc.VectorSubcoreMesh(core_axis_name="c", subcore_axis_name="s"),
        out_type=[jax.ShapeDtypeStruct((NM * K_TOP, VD), F32),
                  jax.ShapeDtypeStruct((NM * K_TOP,), I32)],
        scratch_types=[pltpu.VMEM((512,), I32),
                       pltpu.VMEM((512, VD), F32),
                       pltpu.VMEM((512,), I32),
                       pltpu.SemaphoreType.DMA,
                       pltpu.SemaphoreType.DMA],
    )
    def k(tid_hbm, vals_hbm, eids_hbm, out_v, out_e,
          idx_v, rows_v, eid_v, sem, sem2):
        w = _wid()
        base = w * 512
        pltpu.sync_copy(tid_hbm.at[pl.ds(base, 512)], idx_v)
        cps = []
        for c in range(4):
            sl = pl.ds(c * 128, 128)
            cps.append(pltpu.async_copy(vals_hbm.at[idx_v.at[sl]],
                                        rows_v.at[sl], sem))
            cps.append(pltpu.async_copy(eids_hbm.at[idx_v.at[sl]],
                                        eid_v.at[sl], sem2))
        for cp in cps:
            cp.wait()
        pltpu.sync_copy(rows_v, out_v.at[pl.ds(base, 512)])
        pltpu.sync_copy(eid_v, out_e.at[pl.ds(base, 512)])

    return k


def _abc_body(s_ref, e_ref, wq_ref, bq_ref, k_ref,
              q_out, ts_ref, ti_ref, at_ref,
              q_s, rm_s, code_s):
    pid = pl.program_id(0)

    @pl.when(pid == 0)
    def _():
        cat = jnp.concatenate((s_ref[...], e_ref[...]), axis=-1).astype(BF16)
        wb = wq_ref[...].astype(BF16)
        acc = None
        for i in range(6):
            c = lax.dot_general(cat[:, 256 * i:256 * (i + 1)],
                                wb[256 * i:256 * (i + 1)],
                                (((1,), (0,)), ((), ())),
                                preferred_element_type=F32)
            acc = c if acc is None else acc + c
        q = acc + bq_ref[...]
        q_out[...] = q
        q_s[...] = q

    kb = k_ref[...].astype(BF16)
    qb = q_s[...].astype(BF16)
    st = lax.dot_general(kb, qb, (((1,), (1,)), ((), ())),
                         preferred_element_type=F32)
    s3 = st.reshape(RPB, VPR, NM)
    m = jnp.max(s3, axis=1)
    iot = lax.broadcasted_iota(I32, (RPB, VPR, NM), 1)
    a = jnp.min(jnp.where(s3 == m[:, None, :], iot, VPR), axis=1)
    rowg = pid * RPB + lax.broadcasted_iota(I32, (RPB, NM), 0)
    rm_s[pl.ds(pid * RPB, RPB), :] = m
    code_s[pl.ds(pid * RPB, RPB), :] = rowg * VPR + a

    @pl.when(pid == NBLK - 1)
    def _():
        work = rm_s[...]
        code = code_s[...]
        ts_rows = []
        for k in range(K_TOP):
            mx = jnp.max(work, axis=0, keepdims=True)
            tid = jnp.min(jnp.where(work == mx, code, MSIZE),
                          axis=0, keepdims=True)
            hit = (code >> 6) == (tid >> 6)
            ts_ref[k:k + 1, :] = mx
            ti_ref[k:k + 1, :] = tid
            work = jnp.where(hit, -jnp.inf, work)
            ts_rows.append(mx)
        ts = jnp.concatenate(ts_rows, axis=0)
        ex = jnp.exp(ts - ts[0:1, :])
        at_ref[...] = ex / jnp.sum(ex, axis=0, keepdims=True)


def _e_body(att_ref, v_ref, wu_ref, bu_ref, m_ref,
            enc_ref, bp_ref, sp_ref, ep_ref, g_ref, b_ref,
            o_ref, proj_s):
    pid = pl.program_id(0)

    @pl.when(pid == 0)
    def _():
        v3 = v_ref[...].reshape(NM, K_TOP, VD)
        pooled = jnp.sum(v3 * att_ref[...][:, :, None], axis=1)
        proj = lax.dot_general(
            pooled.astype(BF16), wu_ref[...].astype(BF16),
            (((1,), (0,)), ((), ())), preferred_element_type=F32) + bu_ref[...]
        proj_s[...] = (proj * m_ref[...]).astype(BF16)

    base = pid * ETILE
    r = base + lax.broadcasted_iota(I32, (ETILE, 1), 0)
    ps = bp_ref[...] * T + sp_ref[...]
    pe = bp_ref[...] * T + ep_ref[...]
    mhot = ((r == ps).astype(F32) + (r == pe).astype(F32)).astype(BF16)
    delta = lax.dot_general(mhot, proj_s[...], (((1,), (0,)), ((), ())),
                            preferred_element_type=F32)
    x = enc_ref[...] + delta
    mean = jnp.mean(x, axis=-1, keepdims=True)
    xc = x - mean
    var = jnp.mean(xc * xc, axis=-1, keepdims=True)
    o_ref[...] = xc * lax.rsqrt(var + LN_EPS) * g_ref[...] + b_ref[...]


def _tc_abc(start_enc, end_enc, W_query, b_query, kflat):
    return pl.pallas_call(
        _abc_body,
        grid=(NBLK,),
        in_specs=[pl.BlockSpec((NM, H), lambda k: (0, 0)),
                  pl.BlockSpec((NM, H), lambda k: (0, 0)),
                  pl.BlockSpec((2 * H, KD), lambda k: (0, 0)),
                  pl.BlockSpec((1, KD), lambda k: (0, 0)),
                  pl.BlockSpec((RK, KD), lambda k: (k, 0))],
        out_specs=[pl.BlockSpec((NM, KD), lambda k: (0, 0)),
                   pl.BlockSpec((K_TOP, NM), lambda k: (0, 0)),
                   pl.BlockSpec((K_TOP, NM), lambda k: (0, 0)),
                   pl.BlockSpec((K_TOP, NM), lambda k: (0, 0))],
        out_shape=[jax.ShapeDtypeStruct((NM, KD), F32),
                   jax.ShapeDtypeStruct((K_TOP, NM), F32),
                   jax.ShapeDtypeStruct((K_TOP, NM), I32),
                   jax.ShapeDtypeStruct((K_TOP, NM), F32)],
        scratch_shapes=[pltpu.VMEM((NM, KD), F32),
                        pltpu.VMEM((ROWS, NM), F32),
                        pltpu.VMEM((ROWS, NM), I32)],
    )(start_enc, end_enc, W_query, b_query.reshape(1, KD), kflat)


def _tc_e(attn, values_g, W_update, b_update, mask_f, encf, bp, sp, ep,
          ln_scale, ln_bias):
    return pl.pallas_call(
        _e_body,
        grid=(NTILE,),
        in_specs=[pl.BlockSpec((NM, K_TOP), lambda k: (0, 0)),
                  pl.BlockSpec((NM * K_TOP, VD), lambda k: (0, 0)),
                  pl.BlockSpec((VD, H), lambda k: (0, 0)),
                  pl.BlockSpec((1, H), lambda k: (0, 0)),
                  pl.BlockSpec((NM, 1), lambda k: (0, 0)),
                  pl.BlockSpec((ETILE, H), lambda k: (k, 0)),
                  pl.BlockSpec((1, NM), lambda k: (0, 0)),
                  pl.BlockSpec((1, NM), lambda k: (0, 0)),
                  pl.BlockSpec((1, NM), lambda k: (0, 0)),
                  pl.BlockSpec((1, H), lambda k: (0, 0)),
                  pl.BlockSpec((1, H), lambda k: (0, 0))],
        out_specs=pl.BlockSpec((ETILE, H), lambda k: (k, 0)),
        out_shape=jax.ShapeDtypeStruct((FLAT, H), F32),
        scratch_shapes=[pltpu.VMEM((NM, H), BF16)],
    )(attn, values_g, W_update, b_update.reshape(1, H), mask_f,
      encf, bp, sp, ep, ln_scale.reshape(1, H), ln_bias.reshape(1, H))


def kernel(encoded_input, mention_batch_positions, mention_start_positions,
           mention_end_positions, mention_mask, memory_keys, memory_identifiers,
           memory_entity_ids, memory_values, W_query, b_query, W_update, b_update,
           ln_scale, ln_bias):
    encf = encoded_input.reshape(FLAT, H)
    kflat = memory_keys.reshape(MSIZE, KD)

    start_enc, end_enc = _sc_gather_se()(
        encf, mention_batch_positions, mention_start_positions,
        mention_end_positions)
    queries, ts, ti, attn_t = _tc_abc(start_enc, end_enc, W_query, b_query,
                                      kflat)

    tid_flat = ti.T.reshape(NM * K_TOP)
    values_g, eids = _sc_gather_topk()(tid_flat, memory_values,
                                       memory_entity_ids)

    attn = attn_t.T
    mask_f = mention_mask.astype(F32).reshape(NM, 1)
    enc_out = _tc_e(attn, values_g, W_update, b_update, mask_f, encf,
                    mention_batch_positions.reshape(1, NM),
                    mention_start_positions.reshape(1, NM),
                    mention_end_positions.reshape(1, NM),
                    ln_scale, ln_bias)

    return (enc_out.reshape(B, T, H), queries, attn,
            eids.reshape(NM, K_TOP))


# speedup vs baseline: 4.9850x; 1.0686x over previous
import functools

import jax
import jax.numpy as jnp
from jax import lax
from jax.experimental import pallas as pl
from jax.experimental.pallas import tpu as pltpu
from jax.experimental.pallas import tpu_sc as plsc

F32 = jnp.float32
BF16 = jnp.bfloat16
I32 = jnp.int32

K_TOP = 32
LN_EPS = 1e-12

B, T, H = 4, 2048, 768
NM = 512
ROWS, VPR, KD = 1024, 64, 128
MSIZE = ROWS * VPR
VD = 128
FLAT = B * T
NW = 32
RK = 4096
NBLK = MSIZE // RK
RPB = RK // VPR
ETILE = 512
NTILE = FLAT // ETILE

def _wid():
    return lax.axis_index("s") * 2 + lax.axis_index("c")


@functools.cache
def _sc_gather_se():
    @functools.partial(
        pl.kernel,
        mesh=plsc.VectorSubcoreMesh(core_axis_name="c", subcore_axis_name="s"),
        out_type=[jax.ShapeDtypeStruct((NM, H), F32),
                  jax.ShapeDtypeStruct((NM, H), F32)],
        scratch_types=[pltpu.VMEM((16,), I32),
                       pltpu.VMEM((16,), I32),
                       pltpu.VMEM((16,), I32),
                       pltpu.VMEM((16, H), F32),
                       pltpu.VMEM((16, H), F32),
                       pltpu.SemaphoreType.DMA],
    )
    def k(flat_hbm, bpos_hbm, spos_hbm, epos_hbm, out_s, out_e,
          bidx_v, idx_v, idx2_v, rows_v, rows2_v, sem):
        base = _wid() * 16
        pltpu.sync_copy(bpos_hbm.at[pl.ds(base, 16)], bidx_v)
        pltpu.sync_copy(spos_hbm.at[pl.ds(base, 16)], idx_v)
        pltpu.sync_copy(epos_hbm.at[pl.ds(base, 16)], idx2_v)
        idx_v[...] = bidx_v[...] * T + idx_v[...]
        idx2_v[...] = bidx_v[...] * T + idx2_v[...]
        c1 = pltpu.async_copy(flat_hbm.at[idx_v], rows_v, sem)
        c2 = pltpu.async_copy(flat_hbm.at[idx2_v], rows2_v, sem)
        c1.wait()
        c2.wait()
        pltpu.sync_copy(rows_v, out_s.at[pl.ds(base, 16)])
        pltpu.sync_copy(rows2_v, out_e.at[pl.ds(base, 16)])

    return k


@functools.cache
def _sc_gather_topk():
    @functools.partial(
        pl.kernel,
        mesh=plsc.VectorSubcoreMesh(core_axis_name="c", subcore_axis_name="s"),
        out_type=[jax.ShapeDtypeStruct((NM * K_TOP, VD), F32),
                  jax.ShapeDtypeStruct((NM * K_TOP,), I32)],
        scratch_types=[pltpu.VMEM((512,), I32),
                       pltpu.VMEM((512, VD), F32),
                       pltpu.VMEM((512,), I32),
                       pltpu.SemaphoreType.DMA,
                       pltpu.SemaphoreType.DMA],
    )
    def k(tid_hbm, vals_hbm, eids_hbm, out_v, out_e,
          idx_v, rows_v, eid_v, sem, sem2):
        w = _wid()
        base = w * 512
        pltpu.sync_copy(tid_hbm.at[pl.ds(base, 512)], idx_v)
        cps = []
        for c in range(4):
            sl = pl.ds(c * 128, 128)
            cps.append(pltpu.async_copy(vals_hbm.at[idx_v.at[sl]],
                                        rows_v.at[sl], sem))
            cps.append(pltpu.async_copy(eids_hbm.at[idx_v.at[sl]],
                                        eid_v.at[sl], sem2))
        for cp in cps:
            cp.wait()
        pltpu.sync_copy(rows_v, out_v.at[pl.ds(base, 512)])
        pltpu.sync_copy(eid_v, out_e.at[pl.ds(base, 512)])

    return k


def _abc_body(s_ref, e_ref, wq_ref, bq_ref, k_ref,
              q_out, ts_ref, ti_ref, at_ref,
              q_s, rm_s, code_s):
    pid = pl.program_id(0)

    @pl.when(pid == 0)
    def _():
        cat = jnp.concatenate((s_ref[...], e_ref[...]), axis=-1).astype(BF16)
        wb = wq_ref[...].astype(BF16)
        acc = None
        for i in range(6):
            c = lax.dot_general(cat[:, 256 * i:256 * (i + 1)],
                                wb[256 * i:256 * (i + 1)],
                                (((1,), (0,)), ((), ())),
                                preferred_element_type=F32)
            acc = c if acc is None else acc + c
        q = acc + bq_ref[...]
        q_out[...] = q
        q_s[...] = q

    kb = k_ref[...].astype(BF16)
    qb = q_s[...].astype(BF16)
    st = lax.dot_general(kb, qb, (((1,), (1,)), ((), ())),
                         preferred_element_type=F32)
    s3 = st.reshape(RPB, VPR, NM)
    m = jnp.max(s3, axis=1)
    iot = lax.broadcasted_iota(I32, (RPB, VPR, NM), 1)
    a = jnp.min(jnp.where(s3 == m[:, None, :], iot, VPR), axis=1)
    rowg = pid * RPB + lax.broadcasted_iota(I32, (RPB, NM), 0)
    rm_s[pl.ds(pid * RPB, RPB), :] = m
    code_s[pl.ds(pid * RPB, RPB), :] = rowg * VPR + a

    @pl.when(pid == NBLK - 1)
    def _():
        work = rm_s[...]
        code = code_s[...]
        ts_rows = []
        for k in range(K_TOP):
            mx = jnp.max(work, axis=0, keepdims=True)
            tid = jnp.min(jnp.where(work == mx, code, MSIZE),
                          axis=0, keepdims=True)
            hit = (code >> 6) == (tid >> 6)
            ts_ref[k:k + 1, :] = mx
            ti_ref[k:k + 1, :] = tid
            work = jnp.where(hit, -jnp.inf, work)
            ts_rows.append(mx)
        ts = jnp.concatenate(ts_rows, axis=0)
        ex = jnp.exp(ts - ts[0:1, :])
        at_ref[...] = ex / jnp.sum(ex, axis=0, keepdims=True)


def _e_body(att_ref, v_ref, wu_ref, bu_ref, m_ref,
            enc_ref, bp_ref, sp_ref, ep_ref, g_ref, b_ref,
            o_ref, proj_s):
    pid = pl.program_id(0)

    @pl.when(pid == 0)
    def _():
        v3 = v_ref[...].reshape(NM, K_TOP, VD)
        pooled = jnp.sum(v3 * att_ref[...][:, :, None], axis=1)
        proj = lax.dot_general(
            pooled.astype(BF16), wu_ref[...].astype(BF16),
            (((1,), (0,)), ((), ())), preferred_element_type=F32) + bu_ref[...]
        proj_s[...] = (proj * m_ref[...]).astype(BF16)

    base = pid * ETILE
    r = base + lax.broadcasted_iota(I32, (ETILE, 1), 0)
    ps = bp_ref[...] * T + sp_ref[...]
    pe = bp_ref[...] * T + ep_ref[...]
    mhot = ((r == ps).astype(F32) + (r == pe).astype(F32)).astype(BF16)
    delta = lax.dot_general(mhot, proj_s[...], (((1,), (0,)), ((), ())),
                            preferred_element_type=F32)
    x = enc_ref[...] + delta
    mean = jnp.mean(x, axis=-1, keepdims=True)
    xc = x - mean
    var = jnp.mean(xc * xc, axis=-1, keepdims=True)
    o_ref[...] = xc * lax.rsqrt(var + LN_EPS) * g_ref[...] + b_ref[...]


def _tc_abc(start_enc, end_enc, W_query, b_query, kflat):
    return pl.pallas_call(
        _abc_body,
        grid=(NBLK,),
        in_specs=[pl.BlockSpec((NM, H), lambda k: (0, 0)),
                  pl.BlockSpec((NM, H), lambda k: (0, 0)),
                  pl.BlockSpec((2 * H, KD), lambda k: (0, 0)),
                  pl.BlockSpec((1, KD), lambda k: (0, 0)),
                  pl.BlockSpec((RK, KD), lambda k: (k, 0))],
        out_specs=[pl.BlockSpec((NM, KD), lambda k: (0, 0)),
                   pl.BlockSpec((K_TOP, NM), lambda k: (0, 0)),
                   pl.BlockSpec((K_TOP, NM), lambda k: (0, 0)),
                   pl.BlockSpec((K_TOP, NM), lambda k: (0, 0))],
        out_shape=[jax.ShapeDtypeStruct((NM, KD), F32),
                   jax.ShapeDtypeStruct((K_TOP, NM), F32),
                   jax.ShapeDtypeStruct((K_TOP, NM), I32),
                   jax.ShapeDtypeStruct((K_TOP, NM), F32)],
        scratch_shapes=[pltpu.VMEM((NM, KD), F32),
                        pltpu.VMEM((ROWS, NM), F32),
                        pltpu.VMEM((ROWS, NM), I32)],
    )(start_enc, end_enc, W_query, b_query.reshape(1, KD), kflat)


def _tc_e(attn, values_g, W_update, b_update, mask_f, encf, bp, sp, ep,
          ln_scale, ln_bias):
    return pl.pallas_call(
        _e_body,
        grid=(NTILE,),
        in_specs=[pl.BlockSpec((NM, K_TOP), lambda k: (0, 0)),
                  pl.BlockSpec((NM * K_TOP, VD), lambda k: (0, 0)),
                  pl.BlockSpec((VD, H), lambda k: (0, 0)),
                  pl.BlockSpec((1, H), lambda k: (0, 0)),
                  pl.BlockSpec((NM, 1), lambda k: (0, 0)),
                  pl.BlockSpec((ETILE, H), lambda k: (k, 0)),
                  pl.BlockSpec((1, NM), lambda k: (0, 0)),
                  pl.BlockSpec((1, NM), lambda k: (0, 0)),
                  pl.BlockSpec((1, NM), lambda k: (0, 0)),
                  pl.BlockSpec((1, H), lambda k: (0, 0)),
                  pl.BlockSpec((1, H), lambda k: (0, 0))],
        out_specs=pl.BlockSpec((ETILE, H), lambda k: (k, 0)),
        out_shape=jax.ShapeDtypeStruct((FLAT, H), F32),
        scratch_shapes=[pltpu.VMEM((NM, H), BF16)],
    )(attn, values_g, W_update, b_update.reshape(1, H), mask_f,
      encf, bp, sp, ep, ln_scale.reshape(1, H), ln_bias.reshape(1, H))


def kernel(encoded_input, mention_batch_positions, mention_start_positions,
           mention_end_positions, mention_mask, memory_keys, memory_identifiers,
           memory_entity_ids, memory_values, W_query, b_query, W_update, b_update,
           ln_scale, ln_bias):
    encf = encoded_input.reshape(FLAT, H)
    kflat = memory_keys.reshape(MSIZE, KD)

    start_enc, end_enc = _sc_gather_se()(
        encf, mention_batch_positions, mention_start_positions,
        mention_end_positions)
    queries, ts, ti, attn_t = _tc_abc(start_enc, end_enc, W_query, b_query,
                                      kflat)

    tid_flat = ti.T.reshape(NM * K_TOP)
    values_g, eids = _sc_gather_topk()(tid_flat, memory_values,
                                       memory_entity_ids)

    attn = attn_t.T
    mask_f = mention_mask.astype(F32).reshape(NM, 1)
    enc_out = _tc_e(attn, values_g, W_update, b_update, mask_f, encf,
                    mention_batch_positions.reshape(1, NM),
                    mention_start_positions.reshape(1, NM),
                    mention_end_positions.reshape(1, NM),
                    ln_scale, ln_bias)

    return (enc_out.reshape(B, T, H), queries, attn,
            eids.reshape(NM, K_TOP))


# speedup vs baseline: 5.1311x; 1.0293x over previous
import functools

import jax
import jax.numpy as jnp
from jax import lax
from jax.experimental import pallas as pl
from jax.experimental.pallas import tpu as pltpu
from jax.experimental.pallas import tpu_sc as plsc

F32 = jnp.float32
BF16 = jnp.bfloat16
I32 = jnp.int32

K_TOP = 32
LN_EPS = 1e-12

B, T, H = 4, 2048, 768
NM = 512
ROWS, VPR, KD = 1024, 64, 128
MSIZE = ROWS * VPR
VD = 128
FLAT = B * T
NW = 32
RK = 4096
NBLK = MSIZE // RK
RPB = RK // VPR
ETILE = 1024
NTILE = FLAT // ETILE

def _wid():
    return lax.axis_index("s") * 2 + lax.axis_index("c")


@functools.cache
def _sc_gather_se():
    @functools.partial(
        pl.kernel,
        mesh=plsc.VectorSubcoreMesh(core_axis_name="c", subcore_axis_name="s"),
        out_type=[jax.ShapeDtypeStruct((NM, H), F32),
                  jax.ShapeDtypeStruct((NM, H), F32)],
        scratch_types=[pltpu.VMEM((16,), I32),
                       pltpu.VMEM((16,), I32),
                       pltpu.VMEM((16,), I32),
                       pltpu.VMEM((16, H), F32),
                       pltpu.VMEM((16, H), F32),
                       pltpu.SemaphoreType.DMA],
    )
    def k(flat_hbm, bpos_hbm, spos_hbm, epos_hbm, out_s, out_e,
          bidx_v, idx_v, idx2_v, rows_v, rows2_v, sem):
        base = _wid() * 16
        pltpu.sync_copy(bpos_hbm.at[pl.ds(base, 16)], bidx_v)
        pltpu.sync_copy(spos_hbm.at[pl.ds(base, 16)], idx_v)
        pltpu.sync_copy(epos_hbm.at[pl.ds(base, 16)], idx2_v)
        idx_v[...] = bidx_v[...] * T + idx_v[...]
        idx2_v[...] = bidx_v[...] * T + idx2_v[...]
        c1 = pltpu.async_copy(flat_hbm.at[idx_v], rows_v, sem)
        c2 = pltpu.async_copy(flat_hbm.at[idx2_v], rows2_v, sem)
        c1.wait()
        c2.wait()
        pltpu.sync_copy(rows_v, out_s.at[pl.ds(base, 16)])
        pltpu.sync_copy(rows2_v, out_e.at[pl.ds(base, 16)])

    return k


@functools.cache
def _sc_gather_topk():
    @functools.partial(
        pl.kernel,
        mesh=plsc.VectorSubcoreMesh(core_axis_name="c", subcore_axis_name="s"),
        out_type=[jax.ShapeDtypeStruct((NM * K_TOP, VD), F32),
                  jax.ShapeDtypeStruct((NM * K_TOP,), I32)],
        scratch_types=[pltpu.VMEM((512,), I32),
                       pltpu.VMEM((512, VD), F32),
                       pltpu.VMEM((512,), I32),
                       pltpu.SemaphoreType.DMA,
                       pltpu.SemaphoreType.DMA],
    )
    def k(tid_hbm, vals_hbm, eids_hbm, out_v, out_e,
          idx_v, rows_v, eid_v, sem, sem2):
        w = _wid()
        base = w * 512
        pltpu.sync_copy(tid_hbm.at[pl.ds(base, 512)], idx_v)
        cps = []
        for c in range(4):
            sl = pl.ds(c * 128, 128)
            cps.append(pltpu.async_copy(vals_hbm.at[idx_v.at[sl]],
                                        rows_v.at[sl], sem))
            cps.append(pltpu.async_copy(eids_hbm.at[idx_v.at[sl]],
                                        eid_v.at[sl], sem2))
        for cp in cps:
            cp.wait()
        pltpu.sync_copy(rows_v, out_v.at[pl.ds(base, 512)])
        pltpu.sync_copy(eid_v, out_e.at[pl.ds(base, 512)])

    return k


def _abc_body(s_ref, e_ref, wq_ref, bq_ref, k_ref,
              q_out, ts_ref, ti_ref, at_ref,
              q_s, rm_s, code_s):
    pid = pl.program_id(0)

    @pl.when(pid == 0)
    def _():
        cat = jnp.concatenate((s_ref[...], e_ref[...]), axis=-1).astype(BF16)
        wb = wq_ref[...].astype(BF16)
        acc = None
        for i in range(6):
            c = lax.dot_general(cat[:, 256 * i:256 * (i + 1)],
                                wb[256 * i:256 * (i + 1)],
                                (((1,), (0,)), ((), ())),
                                preferred_element_type=F32)
            acc = c if acc is None else acc + c
        q = acc + bq_ref[...]
        q_out[...] = q
        q_s[...] = q

    kb = k_ref[...].astype(BF16)
    qb = q_s[...].astype(BF16)
    st = lax.dot_general(kb, qb, (((1,), (1,)), ((), ())),
                         preferred_element_type=F32)
    s3 = st.reshape(RPB, VPR, NM)
    m = jnp.max(s3, axis=1)
    iot = lax.broadcasted_iota(I32, (RPB, VPR, NM), 1)
    a = jnp.min(jnp.where(s3 == m[:, None, :], iot, VPR), axis=1)
    rowg = pid * RPB + lax.broadcasted_iota(I32, (RPB, NM), 0)
    rm_s[pl.ds(pid * RPB, RPB), :] = m
    code_s[pl.ds(pid * RPB, RPB), :] = rowg * VPR + a

    @pl.when(pid == NBLK - 1)
    def _():
        work = rm_s[...]
        code = code_s[...]
        ts_rows = []
        for k in range(K_TOP):
            mx = jnp.max(work, axis=0, keepdims=True)
            tid = jnp.min(jnp.where(work == mx, code, MSIZE),
                          axis=0, keepdims=True)
            hit = (code >> 6) == (tid >> 6)
            ts_ref[k:k + 1, :] = mx
            ti_ref[k:k + 1, :] = tid
            work = jnp.where(hit, -jnp.inf, work)
            ts_rows.append(mx)
        ts = jnp.concatenate(ts_rows, axis=0)
        ex = jnp.exp(ts - ts[0:1, :])
        at_ref[...] = ex / jnp.sum(ex, axis=0, keepdims=True)


def _e_body(att_ref, v_ref, wu_ref, bu_ref, m_ref,
            enc_ref, bp_ref, sp_ref, ep_ref, g_ref, b_ref,
            o_ref, proj_s):
    pid = pl.program_id(0)

    @pl.when(pid == 0)
    def _():
        v3 = v_ref[...].reshape(NM, K_TOP, VD)
        pooled = jnp.sum(v3 * att_ref[...][:, :, None], axis=1)
        proj = lax.dot_general(
            pooled.astype(BF16), wu_ref[...].astype(BF16),
            (((1,), (0,)), ((), ())), preferred_element_type=F32) + bu_ref[...]
        proj_s[...] = (proj * m_ref[...]).astype(BF16)

    base = pid * ETILE
    r = base + lax.broadcasted_iota(I32, (ETILE, 1), 0)
    ps = bp_ref[...] * T + sp_ref[...]
    pe = bp_ref[...] * T + ep_ref[...]
    mhot = ((r == ps).astype(F32) + (r == pe).astype(F32)).astype(BF16)
    delta = lax.dot_general(mhot, proj_s[...], (((1,), (0,)), ((), ())),
                            preferred_element_type=F32)
    x = enc_ref[...] + delta
    mean = jnp.mean(x, axis=-1, keepdims=True)
    xc = x - mean
    var = jnp.mean(xc * xc, axis=-1, keepdims=True)
    o_ref[...] = xc * lax.rsqrt(var + LN_EPS) * g_ref[...] + b_ref[...]


def _tc_abc(start_enc, end_enc, W_query, b_query, kflat):
    return pl.pallas_call(
        _abc_body,
        grid=(NBLK,),
        in_specs=[pl.BlockSpec((NM, H), lambda k: (0, 0)),
                  pl.BlockSpec((NM, H), lambda k: (0, 0)),
                  pl.BlockSpec((2 * H, KD), lambda k: (0, 0)),
                  pl.BlockSpec((1, KD), lambda k: (0, 0)),
                  pl.BlockSpec((RK, KD), lambda k: (k, 0))],
        out_specs=[pl.BlockSpec((NM, KD), lambda k: (0, 0)),
                   pl.BlockSpec((K_TOP, NM), lambda k: (0, 0)),
                   pl.BlockSpec((K_TOP, NM), lambda k: (0, 0)),
                   pl.BlockSpec((K_TOP, NM), lambda k: (0, 0))],
        out_shape=[jax.ShapeDtypeStruct((NM, KD), F32),
                   jax.ShapeDtypeStruct((K_TOP, NM), F32),
                   jax.ShapeDtypeStruct((K_TOP, NM), I32),
                   jax.ShapeDtypeStruct((K_TOP, NM), F32)],
        scratch_shapes=[pltpu.VMEM((NM, KD), F32),
                        pltpu.VMEM((ROWS, NM), F32),
                        pltpu.VMEM((ROWS, NM), I32)],
    )(start_enc, end_enc, W_query, b_query.reshape(1, KD), kflat)


def _tc_e(attn, values_g, W_update, b_update, mask_f, encf, bp, sp, ep,
          ln_scale, ln_bias):
    return pl.pallas_call(
        _e_body,
        grid=(NTILE,),
        in_specs=[pl.BlockSpec((NM, K_TOP), lambda k: (0, 0)),
                  pl.BlockSpec((NM * K_TOP, VD), lambda k: (0, 0)),
                  pl.BlockSpec((VD, H), lambda k: (0, 0)),
                  pl.BlockSpec((1, H), lambda k: (0, 0)),
                  pl.BlockSpec((NM, 1), lambda k: (0, 0)),
                  pl.BlockSpec((ETILE, H), lambda k: (k, 0)),
                  pl.BlockSpec((1, NM), lambda k: (0, 0)),
                  pl.BlockSpec((1, NM), lambda k: (0, 0)),
                  pl.BlockSpec((1, NM), lambda k: (0, 0)),
                  pl.BlockSpec((1, H), lambda k: (0, 0)),
                  pl.BlockSpec((1, H), lambda k: (0, 0))],
        out_specs=pl.BlockSpec((ETILE, H), lambda k: (k, 0)),
        out_shape=jax.ShapeDtypeStruct((FLAT, H), F32),
        scratch_shapes=[pltpu.VMEM((NM, H), BF16)],
    )(attn, values_g, W_update, b_update.reshape(1, H), mask_f,
      encf, bp, sp, ep, ln_scale.reshape(1, H), ln_bias.reshape(1, H))


def kernel(encoded_input, mention_batch_positions, mention_start_positions,
           mention_end_positions, mention_mask, memory_keys, memory_identifiers,
           memory_entity_ids, memory_values, W_query, b_query, W_update, b_update,
           ln_scale, ln_bias):
    encf = encoded_input.reshape(FLAT, H)
    kflat = memory_keys.reshape(MSIZE, KD)

    start_enc, end_enc = _sc_gather_se()(
        encf, mention_batch_positions, mention_start_positions,
        mention_end_positions)
    queries, ts, ti, attn_t = _tc_abc(start_enc, end_enc, W_query, b_query,
                                      kflat)

    tid_flat = ti.T.reshape(NM * K_TOP)
    values_g, eids = _sc_gather_topk()(tid_flat, memory_values,
                                       memory_entity_ids)

    attn = attn_t.T
    mask_f = mention_mask.astype(F32).reshape(NM, 1)
    enc_out = _tc_e(attn, values_g, W_update, b_update, mask_f, encf,
                    mention_batch_positions.reshape(1, NM),
                    mention_start_positions.reshape(1, NM),
                    mention_end_positions.reshape(1, NM),
                    ln_scale, ln_bias)

    return (enc_out.reshape(B, T, H), queries, attn,
            eids.reshape(NM, K_TOP))


# speedup vs baseline: 5.1674x; 1.0071x over previous
import functools

import jax
import jax.numpy as jnp
from jax import lax
from jax.experimental import pallas as pl
from jax.experimental.pallas import tpu as pltpu
from jax.experimental.pallas import tpu_sc as plsc

F32 = jnp.float32
BF16 = jnp.bfloat16
I32 = jnp.int32

K_TOP = 32
LN_EPS = 1e-12

B, T, H = 4, 2048, 768
NM = 512
ROWS, VPR, KD = 1024, 64, 128
MSIZE = ROWS * VPR
VD = 128
FLAT = B * T
NW = 32
RK = 4096
NBLK = MSIZE // RK
RPB = RK // VPR
ETILE = 2048
NTILE = FLAT // ETILE

def _wid():
    return lax.axis_index("s") * 2 + lax.axis_index("c")


@functools.cache
def _sc_gather_se():
    @functools.partial(
        pl.kernel,
        mesh=plsc.VectorSubcoreMesh(core_axis_name="c", subcore_axis_name="s"),
        out_type=[jax.ShapeDtypeStruct((NM, H), F32),
                  jax.ShapeDtypeStruct((NM, H), F32)],
        scratch_types=[pltpu.VMEM((16,), I32),
                       pltpu.VMEM((16,), I32),
                       pltpu.VMEM((16,), I32),
                       pltpu.VMEM((16, H), F32),
                       pltpu.VMEM((16, H), F32),
                       pltpu.SemaphoreType.DMA],
    )
    def k(flat_hbm, bpos_hbm, spos_hbm, epos_hbm, out_s, out_e,
          bidx_v, idx_v, idx2_v, rows_v, rows2_v, sem):
        base = _wid() * 16
        pltpu.sync_copy(bpos_hbm.at[pl.ds(base, 16)], bidx_v)
        pltpu.sync_copy(spos_hbm.at[pl.ds(base, 16)], idx_v)
        pltpu.sync_copy(epos_hbm.at[pl.ds(base, 16)], idx2_v)
        idx_v[...] = bidx_v[...] * T + idx_v[...]
        idx2_v[...] = bidx_v[...] * T + idx2_v[...]
        c1 = pltpu.async_copy(flat_hbm.at[idx_v], rows_v, sem)
        c2 = pltpu.async_copy(flat_hbm.at[idx2_v], rows2_v, sem)
        c1.wait()
        c2.wait()
        pltpu.sync_copy(rows_v, out_s.at[pl.ds(base, 16)])
        pltpu.sync_copy(rows2_v, out_e.at[pl.ds(base, 16)])

    return k


@functools.cache
def _sc_gather_topk():
    @functools.partial(
        pl.kernel,
        mesh=plsc.VectorSubcoreMesh(core_axis_name="c", subcore_axis_name="s"),
        out_type=[jax.ShapeDtypeStruct((NM * K_TOP, VD), F32),
                  jax.ShapeDtypeStruct((NM * K_TOP,), I32)],
        scratch_types=[pltpu.VMEM((512,), I32),
                       pltpu.VMEM((512, VD), F32),
                       pltpu.VMEM((512,), I32),
                       pltpu.SemaphoreType.DMA,
                       pltpu.SemaphoreType.DMA],
    )
    def k(tid_hbm, vals_hbm, eids_hbm, out_v, out_e,
          idx_v, rows_v, eid_v, sem, sem2):
        w = _wid()
        base = w * 512
        pltpu.sync_copy(tid_hbm.at[pl.ds(base, 512)], idx_v)
        cps = []
        for c in range(4):
            sl = pl.ds(c * 128, 128)
            cps.append(pltpu.async_copy(vals_hbm.at[idx_v.at[sl]],
                                        rows_v.at[sl], sem))
            cps.append(pltpu.async_copy(eids_hbm.at[idx_v.at[sl]],
                                        eid_v.at[sl], sem2))
        for cp in cps:
            cp.wait()
        pltpu.sync_copy(rows_v, out_v.at[pl.ds(base, 512)])
        pltpu.sync_copy(eid_v, out_e.at[pl.ds(base, 512)])

    return k


def _abc_body(s_ref, e_ref, wq_ref, bq_ref, k_ref,
              q_out, ts_ref, ti_ref, at_ref,
              q_s, rm_s, code_s):
    pid = pl.program_id(0)

    @pl.when(pid == 0)
    def _():
        cat = jnp.concatenate((s_ref[...], e_ref[...]), axis=-1).astype(BF16)
        wb = wq_ref[...].astype(BF16)
        acc = None
        for i in range(6):
            c = lax.dot_general(cat[:, 256 * i:256 * (i + 1)],
                                wb[256 * i:256 * (i + 1)],
                                (((1,), (0,)), ((), ())),
                                preferred_element_type=F32)
            acc = c if acc is None else acc + c
        q = acc + bq_ref[...]
        q_out[...] = q
        q_s[...] = q

    kb = k_ref[...].astype(BF16)
    qb = q_s[...].astype(BF16)
    st = lax.dot_general(kb, qb, (((1,), (1,)), ((), ())),
                         preferred_element_type=F32)
    s3 = st.reshape(RPB, VPR, NM)
    m = jnp.max(s3, axis=1)
    iot = lax.broadcasted_iota(I32, (RPB, VPR, NM), 1)
    a = jnp.min(jnp.where(s3 == m[:, None, :], iot, VPR), axis=1)
    rowg = pid * RPB + lax.broadcasted_iota(I32, (RPB, NM), 0)
    rm_s[pl.ds(pid * RPB, RPB), :] = m
    code_s[pl.ds(pid * RPB, RPB), :] = rowg * VPR + a

    @pl.when(pid == NBLK - 1)
    def _():
        work = rm_s[...]
        code = code_s[...]
        ts_rows = []
        for k in range(K_TOP):
            mx = jnp.max(work, axis=0, keepdims=True)
            tid = jnp.min(jnp.where(work == mx, code, MSIZE),
                          axis=0, keepdims=True)
            hit = (code >> 6) == (tid >> 6)
            ts_ref[k:k + 1, :] = mx
            ti_ref[k:k + 1, :] = tid
            work = jnp.where(hit, -jnp.inf, work)
            ts_rows.append(mx)
        ts = jnp.concatenate(ts_rows, axis=0)
        ex = jnp.exp(ts - ts[0:1, :])
        at_ref[...] = ex / jnp.sum(ex, axis=0, keepdims=True)


def _e_body(att_ref, v_ref, wu_ref, bu_ref, m_ref,
            enc_ref, bp_ref, sp_ref, ep_ref, g_ref, b_ref,
            o_ref, proj_s):
    pid = pl.program_id(0)

    @pl.when(pid == 0)
    def _():
        v3 = v_ref[...].reshape(NM, K_TOP, VD)
        pooled = jnp.sum(v3 * att_ref[...][:, :, None], axis=1)
        proj = lax.dot_general(
            pooled.astype(BF16), wu_ref[...].astype(BF16),
            (((1,), (0,)), ((), ())), preferred_element_type=F32) + bu_ref[...]
        proj_s[...] = (proj * m_ref[...]).astype(BF16)

    base = pid * ETILE
    r = base + lax.broadcasted_iota(I32, (ETILE, 1), 0)
    ps = bp_ref[...] * T + sp_ref[...]
    pe = bp_ref[...] * T + ep_ref[...]
    mhot = ((r == ps).astype(F32) + (r == pe).astype(F32)).astype(BF16)
    delta = lax.dot_general(mhot, proj_s[...], (((1,), (0,)), ((), ())),
                            preferred_element_type=F32)
    x = enc_ref[...] + delta
    mean = jnp.mean(x, axis=-1, keepdims=True)
    xc = x - mean
    var = jnp.mean(xc * xc, axis=-1, keepdims=True)
    o_ref[...] = xc * lax.rsqrt(var + LN_EPS) * g_ref[...] + b_ref[...]


def _tc_abc(start_enc, end_enc, W_query, b_query, kflat):
    return pl.pallas_call(
        _abc_body,
        grid=(NBLK,),
        in_specs=[pl.BlockSpec((NM, H), lambda k: (0, 0)),
                  pl.BlockSpec((NM, H), lambda k: (0, 0)),
                  pl.BlockSpec((2 * H, KD), lambda k: (0, 0)),
                  pl.BlockSpec((1, KD), lambda k: (0, 0)),
                  pl.BlockSpec((RK, KD), lambda k: (k, 0))],
        out_specs=[pl.BlockSpec((NM, KD), lambda k: (0, 0)),
                   pl.BlockSpec((K_TOP, NM), lambda k: (0, 0)),
                   pl.BlockSpec((K_TOP, NM), lambda k: (0, 0)),
                   pl.BlockSpec((K_TOP, NM), lambda k: (0, 0))],
        out_shape=[jax.ShapeDtypeStruct((NM, KD), F32),
                   jax.ShapeDtypeStruct((K_TOP, NM), F32),
                   jax.ShapeDtypeStruct((K_TOP, NM), I32),
                   jax.ShapeDtypeStruct((K_TOP, NM), F32)],
        scratch_shapes=[pltpu.VMEM((NM, KD), F32),
                        pltpu.VMEM((ROWS, NM), F32),
                        pltpu.VMEM((ROWS, NM), I32)],
    )(start_enc, end_enc, W_query, b_query.reshape(1, KD), kflat)


def _tc_e(attn, values_g, W_update, b_update, mask_f, encf, bp, sp, ep,
          ln_scale, ln_bias):
    return pl.pallas_call(
        _e_body,
        grid=(NTILE,),
        in_specs=[pl.BlockSpec((NM, K_TOP), lambda k: (0, 0)),
                  pl.BlockSpec((NM * K_TOP, VD), lambda k: (0, 0)),
                  pl.BlockSpec((VD, H), lambda k: (0, 0)),
                  pl.BlockSpec((1, H), lambda k: (0, 0)),
                  pl.BlockSpec((NM, 1), lambda k: (0, 0)),
                  pl.BlockSpec((ETILE, H), lambda k: (k, 0)),
                  pl.BlockSpec((1, NM), lambda k: (0, 0)),
                  pl.BlockSpec((1, NM), lambda k: (0, 0)),
                  pl.BlockSpec((1, NM), lambda k: (0, 0)),
                  pl.BlockSpec((1, H), lambda k: (0, 0)),
                  pl.BlockSpec((1, H), lambda k: (0, 0))],
        out_specs=pl.BlockSpec((ETILE, H), lambda k: (k, 0)),
        out_shape=jax.ShapeDtypeStruct((FLAT, H), F32),
        scratch_shapes=[pltpu.VMEM((NM, H), BF16)],
    )(attn, values_g, W_update, b_update.reshape(1, H), mask_f,
      encf, bp, sp, ep, ln_scale.reshape(1, H), ln_bias.reshape(1, H))


def kernel(encoded_input, mention_batch_positions, mention_start_positions,
           mention_end_positions, mention_mask, memory_keys, memory_identifiers,
           memory_entity_ids, memory_values, W_query, b_query, W_update, b_update,
           ln_scale, ln_bias):
    encf = encoded_input.reshape(FLAT, H)
    kflat = memory_keys.reshape(MSIZE, KD)

    start_enc, end_enc = _sc_gather_se()(
        encf, mention_batch_positions, mention_start_positions,
        mention_end_positions)
    queries, ts, ti, attn_t = _tc_abc(start_enc, end_enc, W_query, b_query,
                                      kflat)

    tid_flat = ti.T.reshape(NM * K_TOP)
    values_g, eids = _sc_gather_topk()(tid_flat, memory_values,
                                       memory_entity_ids)

    attn = attn_t.T
    mask_f = mention_mask.astype(F32).reshape(NM, 1)
    enc_out = _tc_e(attn, values_g, W_update, b_update, mask_f, encf,
                    mention_batch_positions.reshape(1, NM),
                    mention_start_positions.reshape(1, NM),
                    mention_end_positions.reshape(1, NM),
                    ln_scale, ln_bias)

    return (enc_out.reshape(B, T, H), queries, attn,
            eids.reshape(NM, K_TOP))
